# Initial kernel scaffold; baseline (speedup 1.0000x reference)
import jax
import jax.numpy as jnp
from jax.experimental import pallas as pl


def kernel(xyz, vertices, feature_map, weights, bias, Wkv, bkv, Wq, bq):
    raise NotImplementedError("write your pallas kernel here")


# speedup vs baseline: 10.1677x; 10.1677x over previous
import functools
import math

import jax
import jax.numpy as jnp
from jax import lax
from jax.experimental import pallas as pl
from jax.experimental.pallas import tpu as pltpu
from jax.experimental.pallas import tpu_sc as plsc

_BS, _V, _NS = 8, 2048, 16
_IN, _OUT, _SUP = 128, 64, 2
_BC = max(32, _IN // 2)
_R = 256
_T = _V // _R
_K = _NS + 1
_HALF = _OUT // 2
_VB = 64
_NCHUNK = (_V // 2) // _VB


def _tc_body(vt_ref, vr_ref, fm_ref, wp_ref, bp_ref, a3_ref, c3_ref,
             idx_ref, th_ref, g0_ref, g1_ref):
    vtf = vt_ref[0]
    vtr = vr_ref[0]

    inner = lax.dot_general(vtr, vtf, (((1,), (0,)), ((), ())),
                            preferred_element_type=jnp.float32)
    qc = jnp.sum(vtf * vtf, axis=0, keepdims=True)
    qr = jnp.sum(vtr * vtr, axis=1, keepdims=True)
    dist = (-2.0 * inner + qc) + qr

    a3 = a3_ref[0:3, 0:3]
    c3 = c3_ref[0:1, 0:3]
    bvt = lax.dot_general(a3, vtf, (((1,), (0,)), ((), ())),
                          preferred_element_type=jnp.float32)
    cw = lax.dot_general(c3, vtf, (((1,), (0,)), ((), ())),
                         preferred_element_type=jnp.float32)
    lmat = lax.dot_general(vtr, bvt, (((1,), (0,)), ((), ())),
                           preferred_element_type=jnp.float32) + cw

    iota_l = lax.broadcasted_iota(jnp.int32, (_R, _V), 1)
    dcur = dist
    idx_cols = []
    lg_cols = []
    for k in range(_K):
        m = jnp.min(dcur, axis=1, keepdims=True)
        j = jnp.min(jnp.where(dcur == m, iota_l, _V), axis=1,
                    keepdims=True)
        sel = iota_l == j
        if k > 0:
            lg = jnp.max(jnp.where(sel, lmat, -jnp.inf), axis=1,
                         keepdims=True)
            idx_cols.append(j)
            lg_cols.append(lg)
        dcur = jnp.where(sel, jnp.inf, dcur)

    idxs = jnp.concatenate(idx_cols, axis=1)
    lgs = jnp.concatenate(lg_cols, axis=1) * (1.0 / math.sqrt(_BC))
    tm = jnp.max(lgs, axis=1, keepdims=True)
    ex = jnp.exp(lgs - tm)
    theta = ex / jnp.sum(ex, axis=1, keepdims=True)

    gfull = lax.dot_general(fm_ref[0], wp_ref[...], (((1,), (0,)), ((), ())),
                            preferred_element_type=jnp.float32) + bp_ref[...]
    idx_ref[0] = idxs
    th_ref[0] = theta
    g0_ref[0] = gfull[:, :_HALF]
    g1_ref[0] = gfull[:, _HALF:]


def _tc_stage(vt, vertices, fm, wp, bp, a3p, c3p):
    grid = (_BS, _T)
    return pl.pallas_call(
        _tc_body,
        grid=grid,
        in_specs=[
            pl.BlockSpec((1, 3, _V), lambda b, t: (b, 0, 0)),
            pl.BlockSpec((1, _R, 3), lambda b, t: (b, t, 0)),
            pl.BlockSpec((1, _R, _IN), lambda b, t: (b, t, 0)),
            pl.BlockSpec((_IN, _OUT), lambda b, t: (0, 0)),
            pl.BlockSpec((1, _OUT), lambda b, t: (0, 0)),
            pl.BlockSpec((8, 128), lambda b, t: (0, 0)),
            pl.BlockSpec((8, 128), lambda b, t: (0, 0)),
        ],
        out_specs=[
            pl.BlockSpec((1, _R, _NS), lambda b, t: (b, t, 0)),
            pl.BlockSpec((1, _R, _NS), lambda b, t: (b, t, 0)),
            pl.BlockSpec((1, _R, _HALF), lambda b, t: (b, t, 0)),
            pl.BlockSpec((1, _R, _HALF), lambda b, t: (b, t, 0)),
        ],
        out_shape=[
            jax.ShapeDtypeStruct((_BS, _V, _NS), jnp.int32),
            jax.ShapeDtypeStruct((_BS, _V, _NS), jnp.float32),
            jax.ShapeDtypeStruct((_BS, _V, _HALF), jnp.float32),
            jax.ShapeDtypeStruct((_BS, _V, _HALF), jnp.float32),
        ],
    )(vt, vertices, fm, wp, bp, a3p, c3p)


def _sc_body(g01, idxf, thf, out, gloc, idxloc, thloc, stage):
    cid = lax.axis_index("c")
    sid = lax.axis_index("s")
    wid = sid * 2 + cid
    b = wid // 4
    h = (wid // 2) % 2
    vh = wid % 2
    v0 = vh * (_V // 2)

    pltpu.sync_copy(g01.at[h * _BS + b], gloc)

    def chunk_body(ch, carry):
        e0 = (b * _V + v0 + ch * _VB) * _NS
        pltpu.sync_copy(idxf.at[pl.ds(e0, _VB * _NS)], idxloc)
        pltpu.sync_copy(thf.at[pl.ds(e0, _VB * _NS)], thloc)

        def vb_body(vb, carry2):
            iv = idxloc[pl.ds(vb * _NS, _NS)]
            tv = thloc[pl.ds(vb * _NS, _NS)]
            ivh = iv * _HALF

            def c_body(c, carry3):
                col = plsc.load_gather(gloc, [ivh + c])
                stage[pl.ds(c * (_VB * _NS) + vb * _NS, _NS)] = col * tv
                return carry3

            return lax.fori_loop(0, _HALF, c_body, carry2)

        lax.fori_loop(0, _VB, vb_body, 0)

        def o_body(c, carry2):
            pltpu.sync_copy(stage.at[pl.ds(c * (_VB * _NS), _VB * _NS)],
                            out.at[b * _OUT + h * _HALF + c,
                                   pl.ds((v0 + ch * _VB) * _NS, _VB * _NS)])
            return carry2

        lax.fori_loop(0, _HALF, o_body, 0)
        return carry

    lax.fori_loop(0, _NCHUNK, chunk_body, 0)


def _sc_stage(g01, idxf, thf):
    mesh = plsc.VectorSubcoreMesh(core_axis_name="c", subcore_axis_name="s")
    k = functools.partial(
        pl.kernel,
        out_type=jax.ShapeDtypeStruct((_BS * _OUT, _V * _NS), jnp.float32),
        mesh=mesh,
        compiler_params=pltpu.CompilerParams(needs_layout_passes=False),
        scratch_types=[
            pltpu.VMEM((_V * _HALF,), jnp.float32),
            pltpu.VMEM((_VB * _NS,), jnp.int32),
            pltpu.VMEM((_VB * _NS,), jnp.float32),
            pltpu.VMEM((_HALF * _VB * _NS,), jnp.float32),
        ],
    )(_sc_body)
    return k(g01, idxf, thf)


def kernel(xyz, vertices, feature_map, weights, bias, Wkv, bkv, Wq, bq):
    wp = 0.5 * (weights[:, _OUT:2 * _OUT] + weights[:, 2 * _OUT:])
    bp = (0.5 * (bias[_OUT:2 * _OUT] + bias[2 * _OUT:])).reshape(1, _OUT)
    wk = Wkv[:_BC]
    a3 = jnp.transpose(wk.T @ Wq)
    c3 = (bq @ wk).reshape(1, 3)
    a3p = jnp.zeros((8, 128), jnp.float32).at[0:3, 0:3].set(a3)
    c3p = jnp.zeros((8, 128), jnp.float32).at[0:1, 0:3].set(c3)
    vt = jnp.transpose(vertices, (0, 2, 1))

    idx, theta, g0, g1 = _tc_stage(vt, vertices, feature_map, wp, bp, a3p, c3p)

    g01 = jnp.concatenate([g0, g1], axis=0).reshape(2 * _BS, _V * _HALF)
    idxf = idx.reshape(_BS * _V * _NS)
    thf = theta.reshape(_BS * _V * _NS)
    out = _sc_stage(g01, idxf, thf)
    return out.reshape(_BS, _OUT, _V, _NS)


# speedup vs baseline: 12.6024x; 1.2395x over previous
import functools
import math

import jax
import jax.numpy as jnp
from jax import lax
from jax.experimental import pallas as pl
from jax.experimental.pallas import tpu as pltpu
from jax.experimental.pallas import tpu_sc as plsc

_BS, _V, _NS = 8, 2048, 16
_IN, _OUT, _SUP = 128, 64, 2
_BC = max(32, _IN // 2)
_R = 256
_T = _V // _R
_K = _NS + 1
_HQ = 16
_NQ = _OUT // _HQ
_VB = 128
_NCHUNK = _V // _VB


def _tc_body(vt_ref, vr_ref, fm_ref, wp_ref, bp_ref, a3_ref, c3_ref,
             idx_ref, th_ref, g4_ref):
    vtf = vt_ref[0]
    vtr = vr_ref[0]

    inner = lax.dot_general(vtr, vtf, (((1,), (0,)), ((), ())),
                            preferred_element_type=jnp.float32)
    qc = jnp.sum(vtf * vtf, axis=0, keepdims=True)
    qr = jnp.sum(vtr * vtr, axis=1, keepdims=True)
    dist = (-2.0 * inner + qc) + qr

    a3 = a3_ref[0:3, 0:3]
    c3 = c3_ref[0:1, 0:3]
    bvt = lax.dot_general(a3, vtf, (((1,), (0,)), ((), ())),
                          preferred_element_type=jnp.float32)
    cw = lax.dot_general(c3, vtf, (((1,), (0,)), ((), ())),
                         preferred_element_type=jnp.float32)
    lmat = lax.dot_general(vtr, bvt, (((1,), (0,)), ((), ())),
                           preferred_element_type=jnp.float32) + cw

    iota_l = lax.broadcasted_iota(jnp.int32, (_R, _V), 1)
    dcur = dist
    idx_cols = []
    lg_cols = []
    for k in range(_K):
        m = jnp.min(dcur, axis=1, keepdims=True)
        j = jnp.min(jnp.where(dcur == m, iota_l, _V), axis=1,
                    keepdims=True)
        sel = iota_l == j
        if k > 0:
            lg = jnp.max(jnp.where(sel, lmat, -jnp.inf), axis=1,
                         keepdims=True)
            idx_cols.append(j)
            lg_cols.append(lg)
        dcur = jnp.where(sel, jnp.inf, dcur)

    idxs = jnp.concatenate(idx_cols, axis=1)
    lgs = jnp.concatenate(lg_cols, axis=1) * (1.0 / math.sqrt(_BC))
    tm = jnp.max(lgs, axis=1, keepdims=True)
    ex = jnp.exp(lgs - tm)
    theta = ex / jnp.sum(ex, axis=1, keepdims=True)

    gfull = lax.dot_general(fm_ref[0], wp_ref[...], (((1,), (0,)), ((), ())),
                            preferred_element_type=jnp.float32) + bp_ref[...]
    idx_ref[0] = idxs
    th_ref[0] = theta
    for q in range(_NQ):
        g4_ref[q, 0] = gfull[:, q * _HQ:(q + 1) * _HQ]


def _tc_stage(vt, vertices, fm, wp, bp, a3p, c3p):
    grid = (_BS, _T)
    return pl.pallas_call(
        _tc_body,
        grid=grid,
        in_specs=[
            pl.BlockSpec((1, 3, _V), lambda b, t: (b, 0, 0)),
            pl.BlockSpec((1, _R, 3), lambda b, t: (b, t, 0)),
            pl.BlockSpec((1, _R, _IN), lambda b, t: (b, t, 0)),
            pl.BlockSpec((_IN, _OUT), lambda b, t: (0, 0)),
            pl.BlockSpec((1, _OUT), lambda b, t: (0, 0)),
            pl.BlockSpec((8, 128), lambda b, t: (0, 0)),
            pl.BlockSpec((8, 128), lambda b, t: (0, 0)),
        ],
        out_specs=[
            pl.BlockSpec((1, _R, _NS), lambda b, t: (b, t, 0)),
            pl.BlockSpec((1, _R, _NS), lambda b, t: (b, t, 0)),
            pl.BlockSpec((_NQ, 1, _R, _HQ), lambda b, t: (0, b, t, 0)),
        ],
        out_shape=[
            jax.ShapeDtypeStruct((_BS, _V, _NS), jnp.int32),
            jax.ShapeDtypeStruct((_BS, _V, _NS), jnp.float32),
            jax.ShapeDtypeStruct((_NQ, _BS, _V, _HQ), jnp.float32),
        ],
    )(vt, vertices, fm, wp, bp, a3p, c3p)


def _sc_body(g4, idxf, thf, out, gloc, idxloc, thloc, stage, sems):
    cid = lax.axis_index("c")
    sid = lax.axis_index("s")
    wid = sid * 2 + cid
    b = wid % _BS
    h = wid // _BS
    row_base = b * _OUT + h * _HQ

    pltpu.sync_copy(g4.at[h * _BS + b], gloc)

    def chunk_body(ch, carry):
        slot = lax.rem(ch, 2)
        e0 = (b * _V + ch * _VB) * _NS
        pltpu.sync_copy(idxf.at[pl.ds(e0, _VB * _NS)], idxloc)
        pltpu.sync_copy(thf.at[pl.ds(e0, _VB * _NS)], thloc)

        @pl.when(ch >= 2)
        def _wait_slot():
            pltpu.make_async_copy(
                stage.at[slot],
                out.at[pl.ds(row_base, _HQ), pl.ds(ch * _VB * _NS, _VB * _NS)],
                sems.at[slot]).wait()

        @plsc.parallel_loop(0, _VB)
        def vb_body(vb):
            iv = idxloc[pl.ds(vb * _NS, _NS)]
            tv = thloc[pl.ds(vb * _NS, _NS)]
            ivh = iv * _HQ

            @plsc.parallel_loop(0, _HQ, unroll=8)
            def c_body(c):
                col = plsc.load_gather(gloc, [ivh + c])
                stage[slot, c, pl.ds(vb * _NS, _NS)] = col * tv

        pltpu.make_async_copy(
            stage.at[slot],
            out.at[pl.ds(row_base, _HQ), pl.ds(ch * _VB * _NS, _VB * _NS)],
            sems.at[slot]).start()
        return carry

    lax.fori_loop(0, _NCHUNK, chunk_body, 0)

    for slot in range(2):
        pltpu.make_async_copy(
            stage.at[slot],
            out.at[pl.ds(row_base, _HQ), pl.ds(0, _VB * _NS)],
            sems.at[slot]).wait()


def _sc_stage(g4, idxf, thf):
    mesh = plsc.VectorSubcoreMesh(core_axis_name="c", subcore_axis_name="s")
    k = functools.partial(
        pl.kernel,
        out_type=jax.ShapeDtypeStruct((_BS * _OUT, _V * _NS), jnp.float32),
        mesh=mesh,
        compiler_params=pltpu.CompilerParams(needs_layout_passes=False),
        scratch_types=[
            pltpu.VMEM((_V * _HQ,), jnp.float32),
            pltpu.VMEM((_VB * _NS,), jnp.int32),
            pltpu.VMEM((_VB * _NS,), jnp.float32),
            pltpu.VMEM((2, _HQ, _VB * _NS), jnp.float32),
            pltpu.SemaphoreType.DMA((2,)),
        ],
    )(_sc_body)
    return k(g4, idxf, thf)


def kernel(xyz, vertices, feature_map, weights, bias, Wkv, bkv, Wq, bq):
    wp = 0.5 * (weights[:, _OUT:2 * _OUT] + weights[:, 2 * _OUT:])
    bp = (0.5 * (bias[_OUT:2 * _OUT] + bias[2 * _OUT:])).reshape(1, _OUT)
    wk = Wkv[:_BC]
    a3 = jnp.transpose(wk.T @ Wq)
    c3 = (bq @ wk).reshape(1, 3)
    a3p = jnp.zeros((8, 128), jnp.float32).at[0:3, 0:3].set(a3)
    c3p = jnp.zeros((8, 128), jnp.float32).at[0:1, 0:3].set(c3)
    vt = jnp.transpose(vertices, (0, 2, 1))

    idx, theta, g4 = _tc_stage(vt, vertices, feature_map, wp, bp, a3p, c3p)

    g4f = g4.reshape(_NQ * _BS, _V * _HQ)
    idxf = idx.reshape(_BS * _V * _NS)
    thf = theta.reshape(_BS * _V * _NS)
    out = _sc_stage(g4f, idxf, thf)
    return out.reshape(_BS, _OUT, _V, _NS)


# speedup vs baseline: 13.5900x; 1.0784x over previous
import functools
import math

import jax
import jax.numpy as jnp
from jax import lax
from jax.experimental import pallas as pl
from jax.experimental.pallas import tpu as pltpu
from jax.experimental.pallas import tpu_sc as plsc

_BS, _V, _NS = 8, 2048, 16
_IN, _OUT, _SUP = 128, 64, 2
_BC = max(32, _IN // 2)
_R = 256
_T = _V // _R
_K = _NS + 1
_HQ = 16
_NQ = _OUT // _HQ
_VB = 128
_NCHUNK = _V // _VB


def _tc_body(vt_ref, vr_ref, fm_ref, wp_ref, bp_ref, a3_ref, c3_ref,
             idx_ref, s4_ref, g4_ref):
    vtf = vt_ref[0]
    vtr = vr_ref[0]

    inner = lax.dot_general(vtr, vtf, (((1,), (0,)), ((), ())),
                            preferred_element_type=jnp.float32)
    qc = jnp.sum(vtf * vtf, axis=0, keepdims=True)
    qr = jnp.sum(vtr * vtr, axis=1, keepdims=True)
    dist = (-2.0 * inner + qc) + qr

    a34 = a3_ref[0:3, 0:4]
    c34 = c3_ref[0:1, 0:4]
    s4 = lax.dot_general(vtr, a34, (((1,), (0,)), ((), ())),
                         preferred_element_type=jnp.float32) + c34

    iota_l = lax.broadcasted_iota(jnp.int32, (_R, _V), 1)
    dcur = dist
    idx_cols = []
    for k in range(_K):
        m = jnp.min(dcur, axis=1, keepdims=True)
        j = jnp.min(jnp.where(dcur == m, iota_l, _V), axis=1,
                    keepdims=True)
        if k > 0:
            idx_cols.append(j)
        if k < _K - 1:
            dcur = jnp.where(iota_l == j, jnp.inf, dcur)

    idxs = jnp.concatenate(idx_cols, axis=1)

    gfull = lax.dot_general(fm_ref[0], wp_ref[...], (((1,), (0,)), ((), ())),
                            preferred_element_type=jnp.float32) + bp_ref[...]
    idx_ref[0] = idxs
    s4_ref[0] = s4
    for q in range(_NQ):
        g4_ref[q, 0] = gfull[:, q * _HQ:(q + 1) * _HQ]


def _tc_stage(vt, vertices, fm, wp, bp, a3p, c3p):
    grid = (_BS, _T)
    return pl.pallas_call(
        _tc_body,
        grid=grid,
        in_specs=[
            pl.BlockSpec((1, 3, _V), lambda b, t: (b, 0, 0)),
            pl.BlockSpec((1, _R, 3), lambda b, t: (b, t, 0)),
            pl.BlockSpec((1, _R, _IN), lambda b, t: (b, t, 0)),
            pl.BlockSpec((_IN, _OUT), lambda b, t: (0, 0)),
            pl.BlockSpec((1, _OUT), lambda b, t: (0, 0)),
            pl.BlockSpec((8, 128), lambda b, t: (0, 0)),
            pl.BlockSpec((8, 128), lambda b, t: (0, 0)),
        ],
        out_specs=[
            pl.BlockSpec((1, _R, _NS), lambda b, t: (b, t, 0)),
            pl.BlockSpec((1, _R, 4), lambda b, t: (b, t, 0)),
            pl.BlockSpec((_NQ, 1, _R, _HQ), lambda b, t: (0, b, t, 0)),
        ],
        out_shape=[
            jax.ShapeDtypeStruct((_BS, _V, _NS), jnp.int32),
            jax.ShapeDtypeStruct((_BS, _V, 4), jnp.float32),
            jax.ShapeDtypeStruct((_NQ, _BS, _V, _HQ), jnp.float32),
        ],
    )(vt, vertices, fm, wp, bp, a3p, c3p)


def _sc_body(g4, idxf, s4f, v4, out, gloc, idxloc, sloc, cloc, stage, sems):
    cid = lax.axis_index("c")
    sid = lax.axis_index("s")
    wid = sid * 2 + cid
    b = wid % _BS
    h = wid // _BS
    row_base = b * _OUT + h * _HQ

    pltpu.sync_copy(g4.at[h * _BS + b], gloc)
    pltpu.sync_copy(v4.at[b], cloc)

    def chunk_body(ch, carry):
        slot = lax.rem(ch, 2)
        e0 = (b * _V + ch * _VB) * _NS
        pltpu.sync_copy(idxf.at[pl.ds(e0, _VB * _NS)], idxloc)
        pltpu.sync_copy(s4f.at[pl.ds((b * _V + ch * _VB) * 4, _VB * 4)],
                        sloc.at[pl.ds(0, _VB * 4)])

        @pl.when(ch >= 2)
        def _wait_slot():
            pltpu.make_async_copy(
                stage.at[slot],
                out.at[pl.ds(row_base, _HQ), pl.ds(ch * _VB * _NS, _VB * _NS)],
                sems.at[slot]).wait()

        @plsc.parallel_loop(0, _VB)
        def vb_body(vb):
            iv = idxloc[pl.ds(vb * _NS, _NS)]
            iv4 = iv * 4
            xn = plsc.load_gather(cloc, [iv4])
            yn = plsc.load_gather(cloc, [iv4 + 1])
            zn = plsc.load_gather(cloc, [iv4 + 2])
            sv = sloc[pl.ds(vb * 4, _NS)]
            lg = xn * sv[0] + yn * sv[1] + zn * sv[2]
            mx = jnp.max(lg)
            ex = jnp.exp((lg - mx) * (1.0 / math.sqrt(_BC)))
            tv = ex / jnp.sum(ex)
            ivh = iv * _HQ

            @plsc.parallel_loop(0, _HQ, unroll=8)
            def c_body(c):
                col = plsc.load_gather(gloc, [ivh + c])
                stage[slot, c, pl.ds(vb * _NS, _NS)] = col * tv

        pltpu.make_async_copy(
            stage.at[slot],
            out.at[pl.ds(row_base, _HQ), pl.ds(ch * _VB * _NS, _VB * _NS)],
            sems.at[slot]).start()
        return carry

    lax.fori_loop(0, _NCHUNK, chunk_body, 0)

    for slot in range(2):
        pltpu.make_async_copy(
            stage.at[slot],
            out.at[pl.ds(row_base, _HQ), pl.ds(0, _VB * _NS)],
            sems.at[slot]).wait()


def _sc_stage(g4, idxf, s4f, v4):
    mesh = plsc.VectorSubcoreMesh(core_axis_name="c", subcore_axis_name="s")
    k = functools.partial(
        pl.kernel,
        out_type=jax.ShapeDtypeStruct((_BS * _OUT, _V * _NS), jnp.float32),
        mesh=mesh,
        compiler_params=pltpu.CompilerParams(needs_layout_passes=False),
        scratch_types=[
            pltpu.VMEM((_V * _HQ,), jnp.float32),
            pltpu.VMEM((_VB * _NS,), jnp.int32),
            pltpu.VMEM((_VB * 4 + _NS,), jnp.float32),
            pltpu.VMEM((_V * 4,), jnp.float32),
            pltpu.VMEM((2, _HQ, _VB * _NS), jnp.float32),
            pltpu.SemaphoreType.DMA((2,)),
        ],
    )(_sc_body)
    return k(g4, idxf, s4f, v4)


def kernel(xyz, vertices, feature_map, weights, bias, Wkv, bkv, Wq, bq):
    wp = 0.5 * (weights[:, _OUT:2 * _OUT] + weights[:, 2 * _OUT:])
    bp = (0.5 * (bias[_OUT:2 * _OUT] + bias[2 * _OUT:])).reshape(1, _OUT)
    wk = Wkv[:_BC]
    a3 = jnp.transpose(wk.T @ Wq)
    c3 = (bq @ wk).reshape(1, 3)
    a3p = jnp.zeros((8, 128), jnp.float32).at[0:3, 0:3].set(a3)
    c3p = jnp.zeros((8, 128), jnp.float32).at[0:1, 0:3].set(c3)
    vt = jnp.transpose(vertices, (0, 2, 1))

    idx, s4, g4 = _tc_stage(vt, vertices, feature_map, wp, bp, a3p, c3p)

    g4f = g4.reshape(_NQ * _BS, _V * _HQ)
    idxf = idx.reshape(_BS * _V * _NS)
    s4f = s4.reshape(_BS * _V * 4)
    v4 = jnp.concatenate(
        [vertices, jnp.zeros((_BS, _V, 1), jnp.float32)], axis=2
    ).reshape(_BS, _V * 4)
    out = _sc_stage(g4f, idxf, s4f, v4)
    return out.reshape(_BS, _OUT, _V, _NS)


# speedup vs baseline: 14.6220x; 1.0759x over previous
import functools
import math

import jax
import jax.numpy as jnp
from jax import lax
from jax.experimental import pallas as pl
from jax.experimental.pallas import tpu as pltpu
from jax.experimental.pallas import tpu_sc as plsc

_BS, _V, _NS = 8, 2048, 16
_IN, _OUT, _SUP = 128, 64, 2
_BC = max(32, _IN // 2)
_R = 256
_T = _V // _R
_K = _NS + 1
_HQ = 16
_NQ = _OUT // _HQ
_VB = 128
_NCHUNK = _V // _VB


def _tc_body(vt_ref, vr_ref, fm_ref, wp_ref, bp_ref, a3_ref, c3_ref,
             idx_ref, s4_ref, g4_ref):
    vtf = vt_ref[0]
    vtr = vr_ref[0]

    inner = lax.dot_general(vtr, vtf, (((1,), (0,)), ((), ())),
                            preferred_element_type=jnp.float32)
    qc = jnp.sum(vtf * vtf, axis=0, keepdims=True)
    qr = jnp.sum(vtr * vtr, axis=1, keepdims=True)
    dist = (-2.0 * inner + qc) + qr

    a34 = a3_ref[0:3, 0:4]
    c34 = c3_ref[0:1, 0:4]
    s4 = lax.dot_general(vtr, a34, (((1,), (0,)), ((), ())),
                         preferred_element_type=jnp.float32) + c34

    _NC, _CW = 16, 128
    chunks = [dist[:, c * _CW:(c + 1) * _CW] for c in range(_NC)]
    cm = jnp.concatenate(
        [jnp.min(ch, axis=1, keepdims=True) for ch in chunks], axis=1)
    iota_nc = lax.broadcasted_iota(jnp.int32, (_R, _NC), 1).astype(jnp.float32)
    iota_cw = lax.broadcasted_iota(jnp.int32, (_R, _CW), 1).astype(jnp.float32)
    idx_cols = []
    hist = []
    for k in range(_K):
        m = jnp.min(cm, axis=1, keepdims=True)
        jc = jnp.min(jnp.where(cm == m, iota_nc, float(_NC)), axis=1,
                     keepdims=True)
        jci = jc.astype(jnp.int32)
        lvl = chunks
        for lbit in range(4):
            bit = ((jci >> lbit) & 1) == 1
            lvl = [jnp.where(bit, lvl[2 * i + 1], lvl[2 * i])
                   for i in range(len(lvl) // 2)]
        ext = lvl[0]
        for (pjc, pjl) in hist:
            ext = jnp.where((pjc == jc) & (iota_cw == pjl), jnp.inf, ext)
        jl = jnp.min(jnp.where(ext == m, iota_cw, float(_CW)), axis=1,
                     keepdims=True)
        if k > 0:
            idx_cols.append(jc * float(_CW) + jl)
        if k < _K - 1:
            extm = jnp.where(iota_cw == jl, jnp.inf, ext)
            nm = jnp.min(extm, axis=1, keepdims=True)
            cm = jnp.where(iota_nc == jc, nm, cm)
            hist.append((jc, jl))

    idxs = jnp.concatenate(idx_cols, axis=1).astype(jnp.int32)

    gfull = lax.dot_general(fm_ref[0], wp_ref[...], (((1,), (0,)), ((), ())),
                            preferred_element_type=jnp.float32) + bp_ref[...]
    idx_ref[0] = idxs
    s4_ref[0] = s4
    for q in range(_NQ):
        g4_ref[q, 0] = gfull[:, q * _HQ:(q + 1) * _HQ]


def _tc_stage(vt, vertices, fm, wp, bp, a3p, c3p):
    grid = (_BS, _T)
    return pl.pallas_call(
        _tc_body,
        grid=grid,
        in_specs=[
            pl.BlockSpec((1, 3, _V), lambda b, t: (b, 0, 0)),
            pl.BlockSpec((1, _R, 3), lambda b, t: (b, t, 0)),
            pl.BlockSpec((1, _R, _IN), lambda b, t: (b, t, 0)),
            pl.BlockSpec((_IN, _OUT), lambda b, t: (0, 0)),
            pl.BlockSpec((1, _OUT), lambda b, t: (0, 0)),
            pl.BlockSpec((8, 128), lambda b, t: (0, 0)),
            pl.BlockSpec((8, 128), lambda b, t: (0, 0)),
        ],
        out_specs=[
            pl.BlockSpec((1, _R, _NS), lambda b, t: (b, t, 0)),
            pl.BlockSpec((1, _R, 4), lambda b, t: (b, t, 0)),
            pl.BlockSpec((_NQ, 1, _R, _HQ), lambda b, t: (0, b, t, 0)),
        ],
        out_shape=[
            jax.ShapeDtypeStruct((_BS, _V, _NS), jnp.int32),
            jax.ShapeDtypeStruct((_BS, _V, 4), jnp.float32),
            jax.ShapeDtypeStruct((_NQ, _BS, _V, _HQ), jnp.float32),
        ],
    )(vt, vertices, fm, wp, bp, a3p, c3p)


def _sc_body(g4, idxf, s4f, v4, out, gloc, idxloc, sloc, cloc, stage, sems):
    cid = lax.axis_index("c")
    sid = lax.axis_index("s")
    wid = sid * 2 + cid
    b = wid % _BS
    h = wid // _BS
    row_base = b * _OUT + h * _HQ

    pltpu.sync_copy(g4.at[h * _BS + b], gloc)
    pltpu.sync_copy(v4.at[b], cloc)

    def chunk_body(ch, carry):
        slot = lax.rem(ch, 2)
        e0 = (b * _V + ch * _VB) * _NS
        pltpu.sync_copy(idxf.at[pl.ds(e0, _VB * _NS)], idxloc)
        pltpu.sync_copy(s4f.at[pl.ds((b * _V + ch * _VB) * 4, _VB * 4)],
                        sloc.at[pl.ds(0, _VB * 4)])

        @pl.when(ch >= 2)
        def _wait_slot():
            pltpu.make_async_copy(
                stage.at[slot],
                out.at[pl.ds(row_base, _HQ), pl.ds(ch * _VB * _NS, _VB * _NS)],
                sems.at[slot]).wait()

        @plsc.parallel_loop(0, _VB)
        def vb_body(vb):
            iv = idxloc[pl.ds(vb * _NS, _NS)]
            iv4 = iv * 4
            xn = plsc.load_gather(cloc, [iv4])
            yn = plsc.load_gather(cloc, [iv4 + 1])
            zn = plsc.load_gather(cloc, [iv4 + 2])
            sv = sloc[pl.ds(vb * 4, _NS)]
            lg = xn * sv[0] + yn * sv[1] + zn * sv[2]
            mx = jnp.max(lg)
            ex = jnp.exp((lg - mx) * (1.0 / math.sqrt(_BC)))
            tv = ex / jnp.sum(ex)
            ivh = iv * _HQ

            @plsc.parallel_loop(0, _HQ, unroll=8)
            def c_body(c):
                col = plsc.load_gather(gloc, [ivh + c])
                stage[slot, c, pl.ds(vb * _NS, _NS)] = col * tv

        pltpu.make_async_copy(
            stage.at[slot],
            out.at[pl.ds(row_base, _HQ), pl.ds(ch * _VB * _NS, _VB * _NS)],
            sems.at[slot]).start()
        return carry

    lax.fori_loop(0, _NCHUNK, chunk_body, 0)

    for slot in range(2):
        pltpu.make_async_copy(
            stage.at[slot],
            out.at[pl.ds(row_base, _HQ), pl.ds(0, _VB * _NS)],
            sems.at[slot]).wait()


def _sc_stage(g4, idxf, s4f, v4):
    mesh = plsc.VectorSubcoreMesh(core_axis_name="c", subcore_axis_name="s")
    k = functools.partial(
        pl.kernel,
        out_type=jax.ShapeDtypeStruct((_BS * _OUT, _V * _NS), jnp.float32),
        mesh=mesh,
        compiler_params=pltpu.CompilerParams(needs_layout_passes=False),
        scratch_types=[
            pltpu.VMEM((_V * _HQ,), jnp.float32),
            pltpu.VMEM((_VB * _NS,), jnp.int32),
            pltpu.VMEM((_VB * 4 + _NS,), jnp.float32),
            pltpu.VMEM((_V * 4,), jnp.float32),
            pltpu.VMEM((2, _HQ, _VB * _NS), jnp.float32),
            pltpu.SemaphoreType.DMA((2,)),
        ],
    )(_sc_body)
    return k(g4, idxf, s4f, v4)


def kernel(xyz, vertices, feature_map, weights, bias, Wkv, bkv, Wq, bq):
    wp = 0.5 * (weights[:, _OUT:2 * _OUT] + weights[:, 2 * _OUT:])
    bp = (0.5 * (bias[_OUT:2 * _OUT] + bias[2 * _OUT:])).reshape(1, _OUT)
    wk = Wkv[:_BC]
    a3 = jnp.transpose(wk.T @ Wq)
    c3 = (bq @ wk).reshape(1, 3)
    a3p = jnp.zeros((8, 128), jnp.float32).at[0:3, 0:3].set(a3)
    c3p = jnp.zeros((8, 128), jnp.float32).at[0:1, 0:3].set(c3)
    vt = jnp.transpose(vertices, (0, 2, 1))

    idx, s4, g4 = _tc_stage(vt, vertices, feature_map, wp, bp, a3p, c3p)

    g4f = g4.reshape(_NQ * _BS, _V * _HQ)
    idxf = idx.reshape(_BS * _V * _NS)
    s4f = s4.reshape(_BS * _V * 4)
    v4 = jnp.concatenate(
        [vertices, jnp.zeros((_BS, _V, 1), jnp.float32)], axis=2
    ).reshape(_BS, _V * 4)
    out = _sc_stage(g4f, idxf, s4f, v4)
    return out.reshape(_BS, _OUT, _V, _NS)


# speedup vs baseline: 15.7697x; 1.0785x over previous
import functools
import math

import jax
import jax.numpy as jnp
from jax import lax
from jax.experimental import pallas as pl
from jax.experimental.pallas import tpu as pltpu
from jax.experimental.pallas import tpu_sc as plsc

_BS, _V, _NS = 8, 2048, 16
_IN, _OUT, _SUP = 128, 64, 2
_BC = max(32, _IN // 2)
_R = 512
_T = _V // _R
_K = _NS + 1
_HQ = 16
_NQ = _OUT // _HQ
_VB = 128
_NCHUNK = _V // _VB


def _tc_body(vt_ref, vr_ref, fm_ref, wp_ref, bp_ref, a3_ref, c3_ref,
             idx_ref, s4_ref, g4_ref):
    vtf = vt_ref[0]
    vtr = vr_ref[0]

    inner = lax.dot_general(vtr, vtf, (((1,), (0,)), ((), ())),
                            preferred_element_type=jnp.float32)
    qc = jnp.sum(vtf * vtf, axis=0, keepdims=True)
    qr = jnp.sum(vtr * vtr, axis=1, keepdims=True)
    dist = (-2.0 * inner + qc) + qr

    a34 = a3_ref[0:3, 0:4]
    c34 = c3_ref[0:1, 0:4]
    s4 = lax.dot_general(vtr, a34, (((1,), (0,)), ((), ())),
                         preferred_element_type=jnp.float32) + c34

    _NC, _CW = 16, 128
    chunks = [dist[:, c * _CW:(c + 1) * _CW] for c in range(_NC)]
    cm = jnp.concatenate(
        [jnp.min(ch, axis=1, keepdims=True) for ch in chunks], axis=1)
    iota_nc = lax.broadcasted_iota(jnp.int32, (_R, _NC), 1).astype(jnp.float32)
    iota_cw = lax.broadcasted_iota(jnp.int32, (_R, _CW), 1).astype(jnp.float32)
    idx_cols = []
    hist = []
    for k in range(_K):
        m = jnp.min(cm, axis=1, keepdims=True)
        jc = jnp.min(jnp.where(cm == m, iota_nc, float(_NC)), axis=1,
                     keepdims=True)
        jci = jc.astype(jnp.int32)
        lvl = chunks
        for lbit in range(4):
            bit = ((jci >> lbit) & 1) == 1
            lvl = [jnp.where(bit, lvl[2 * i + 1], lvl[2 * i])
                   for i in range(len(lvl) // 2)]
        ext = lvl[0]
        for (pjc, pjl) in hist:
            ext = jnp.where((pjc == jc) & (iota_cw == pjl), jnp.inf, ext)
        jl = jnp.min(jnp.where(ext == m, iota_cw, float(_CW)), axis=1,
                     keepdims=True)
        if k > 0:
            idx_cols.append(jc * float(_CW) + jl)
        if k < _K - 1:
            extm = jnp.where(iota_cw == jl, jnp.inf, ext)
            nm = jnp.min(extm, axis=1, keepdims=True)
            cm = jnp.where(iota_nc == jc, nm, cm)
            hist.append((jc, jl))

    idxs = jnp.concatenate(idx_cols, axis=1).astype(jnp.int32)

    gfull = lax.dot_general(fm_ref[0], wp_ref[...], (((1,), (0,)), ((), ())),
                            preferred_element_type=jnp.float32) + bp_ref[...]
    idx_ref[0] = idxs
    s4_ref[0] = s4
    for q in range(_NQ):
        g4_ref[q, 0] = gfull[:, q * _HQ:(q + 1) * _HQ]


def _tc_stage(vt, vertices, fm, wp, bp, a3p, c3p):
    grid = (_BS, _T)
    return pl.pallas_call(
        _tc_body,
        grid=grid,
        in_specs=[
            pl.BlockSpec((1, 3, _V), lambda b, t: (b, 0, 0)),
            pl.BlockSpec((1, _R, 3), lambda b, t: (b, t, 0)),
            pl.BlockSpec((1, _R, _IN), lambda b, t: (b, t, 0)),
            pl.BlockSpec((_IN, _OUT), lambda b, t: (0, 0)),
            pl.BlockSpec((1, _OUT), lambda b, t: (0, 0)),
            pl.BlockSpec((8, 128), lambda b, t: (0, 0)),
            pl.BlockSpec((8, 128), lambda b, t: (0, 0)),
        ],
        out_specs=[
            pl.BlockSpec((1, _R, _NS), lambda b, t: (b, t, 0)),
            pl.BlockSpec((1, _R, 4), lambda b, t: (b, t, 0)),
            pl.BlockSpec((_NQ, 1, _R, _HQ), lambda b, t: (0, b, t, 0)),
        ],
        out_shape=[
            jax.ShapeDtypeStruct((_BS, _V, _NS), jnp.int32),
            jax.ShapeDtypeStruct((_BS, _V, 4), jnp.float32),
            jax.ShapeDtypeStruct((_NQ, _BS, _V, _HQ), jnp.float32),
        ],
    )(vt, vertices, fm, wp, bp, a3p, c3p)


def _sc_body(g4, idxf, s4f, v4, out, gloc, idxloc, sloc, cloc, stage, sems):
    cid = lax.axis_index("c")
    sid = lax.axis_index("s")
    wid = sid * 2 + cid
    b = wid % _BS
    h = wid // _BS
    row_base = b * _OUT + h * _HQ

    pltpu.sync_copy(g4.at[h * _BS + b], gloc)
    pltpu.sync_copy(v4.at[b], cloc)

    def chunk_body(ch, carry):
        slot = lax.rem(ch, 2)
        e0 = (b * _V + ch * _VB) * _NS
        pltpu.sync_copy(idxf.at[pl.ds(e0, _VB * _NS)], idxloc)
        pltpu.sync_copy(s4f.at[pl.ds((b * _V + ch * _VB) * 4, _VB * 4)],
                        sloc.at[pl.ds(0, _VB * 4)])

        @pl.when(ch >= 2)
        def _wait_slot():
            pltpu.make_async_copy(
                stage.at[slot],
                out.at[pl.ds(row_base, _HQ), pl.ds(ch * _VB * _NS, _VB * _NS)],
                sems.at[slot]).wait()

        @plsc.parallel_loop(0, _VB)
        def vb_body(vb):
            iv = idxloc[pl.ds(vb * _NS, _NS)]
            iv4 = iv * 4
            xn = plsc.load_gather(cloc, [iv4])
            yn = plsc.load_gather(cloc, [iv4 + 1])
            zn = plsc.load_gather(cloc, [iv4 + 2])
            sv = sloc[pl.ds(vb * 4, _NS)]
            lg = xn * sv[0] + yn * sv[1] + zn * sv[2]
            mx = jnp.max(lg)
            ex = jnp.exp((lg - mx) * (1.0 / math.sqrt(_BC)))
            tv = ex / jnp.sum(ex)
            ivh = iv * _HQ

            @plsc.parallel_loop(0, _HQ, unroll=8)
            def c_body(c):
                col = plsc.load_gather(gloc, [ivh + c])
                stage[slot, c, pl.ds(vb * _NS, _NS)] = col * tv

        pltpu.make_async_copy(
            stage.at[slot],
            out.at[pl.ds(row_base, _HQ), pl.ds(ch * _VB * _NS, _VB * _NS)],
            sems.at[slot]).start()
        return carry

    lax.fori_loop(0, _NCHUNK, chunk_body, 0)

    for slot in range(2):
        pltpu.make_async_copy(
            stage.at[slot],
            out.at[pl.ds(row_base, _HQ), pl.ds(0, _VB * _NS)],
            sems.at[slot]).wait()


def _sc_stage(g4, idxf, s4f, v4):
    mesh = plsc.VectorSubcoreMesh(core_axis_name="c", subcore_axis_name="s")
    k = functools.partial(
        pl.kernel,
        out_type=jax.ShapeDtypeStruct((_BS * _OUT, _V * _NS), jnp.float32),
        mesh=mesh,
        compiler_params=pltpu.CompilerParams(needs_layout_passes=False),
        scratch_types=[
            pltpu.VMEM((_V * _HQ,), jnp.float32),
            pltpu.VMEM((_VB * _NS,), jnp.int32),
            pltpu.VMEM((_VB * 4 + _NS,), jnp.float32),
            pltpu.VMEM((_V * 4,), jnp.float32),
            pltpu.VMEM((2, _HQ, _VB * _NS), jnp.float32),
            pltpu.SemaphoreType.DMA((2,)),
        ],
    )(_sc_body)
    return k(g4, idxf, s4f, v4)


def kernel(xyz, vertices, feature_map, weights, bias, Wkv, bkv, Wq, bq):
    wp = 0.5 * (weights[:, _OUT:2 * _OUT] + weights[:, 2 * _OUT:])
    bp = (0.5 * (bias[_OUT:2 * _OUT] + bias[2 * _OUT:])).reshape(1, _OUT)
    wk = Wkv[:_BC]
    a3 = jnp.transpose(wk.T @ Wq)
    c3 = (bq @ wk).reshape(1, 3)
    a3p = jnp.zeros((8, 128), jnp.float32).at[0:3, 0:3].set(a3)
    c3p = jnp.zeros((8, 128), jnp.float32).at[0:1, 0:3].set(c3)
    vt = jnp.transpose(vertices, (0, 2, 1))

    idx, s4, g4 = _tc_stage(vt, vertices, feature_map, wp, bp, a3p, c3p)

    g4f = g4.reshape(_NQ * _BS, _V * _HQ)
    idxf = idx.reshape(_BS * _V * _NS)
    s4f = s4.reshape(_BS * _V * 4)
    v4 = jnp.concatenate(
        [vertices, jnp.zeros((_BS, _V, 1), jnp.float32)], axis=2
    ).reshape(_BS, _V * 4)
    out = _sc_stage(g4f, idxf, s4f, v4)
    return out.reshape(_BS, _OUT, _V, _NS)


# speedup vs baseline: 17.6702x; 1.1205x over previous
import functools
import math

import jax
import jax.numpy as jnp
from jax import lax
from jax.experimental import pallas as pl
from jax.experimental.pallas import tpu as pltpu
from jax.experimental.pallas import tpu_sc as plsc

_BS, _V, _NS = 8, 2048, 16
_IN, _OUT, _SUP = 128, 64, 2
_BC = max(32, _IN // 2)
_R = 512
_T = _V // _R
_K = _NS + 1
_HQ = 16
_NQ = _OUT // _HQ
_NB = 2
_VB = 128
_NCHUNK = (_V // 4) // _VB


def _tc_body(vt_ref, vr_ref, fm_ref, wp_ref, bp_ref, a3_ref, c3_ref,
             idx_ref, s4_ref, v4_ref, g4_ref):
    vtf = vt_ref[0]
    vtr = vr_ref[0]

    inner = lax.dot_general(vtr, vtf, (((1,), (0,)), ((), ())),
                            preferred_element_type=jnp.float32)
    qc = jnp.sum(vtf * vtf, axis=0, keepdims=True)
    qr = jnp.sum(vtr * vtr, axis=1, keepdims=True)
    dist = (-2.0 * inner + qc) + qr

    a34 = a3_ref[0:3, 0:4]
    c34 = c3_ref[0:1, 0:4]
    s4 = lax.dot_general(vtr, a34, (((1,), (0,)), ((), ())),
                         preferred_element_type=jnp.float32) + c34

    _NC, _CW = 16, 128
    chunks = [dist[:, c * _CW:(c + 1) * _CW] for c in range(_NC)]
    cm = jnp.concatenate(
        [jnp.min(ch, axis=1, keepdims=True) for ch in chunks], axis=1)
    iota_nc = lax.broadcasted_iota(jnp.int32, (_R, _NC), 1).astype(jnp.float32)
    iota_cw = lax.broadcasted_iota(jnp.int32, (_R, _CW), 1).astype(jnp.float32)
    idx_cols = []
    hist = []
    for k in range(_K):
        m = jnp.min(cm, axis=1, keepdims=True)
        jc = jnp.min(jnp.where(cm == m, iota_nc, float(_NC)), axis=1,
                     keepdims=True)
        jci = jc.astype(jnp.int32)
        lvl = chunks
        for lbit in range(4):
            bit = ((jci >> lbit) & 1) == 1
            lvl = [jnp.where(bit, lvl[2 * i + 1], lvl[2 * i])
                   for i in range(len(lvl) // 2)]
        ext = lvl[0]
        for (pjc, pjl) in hist:
            ext = jnp.where((pjc == jc) & (iota_cw == pjl), jnp.inf, ext)
        jl = jnp.min(jnp.where(ext == m, iota_cw, float(_CW)), axis=1,
                     keepdims=True)
        if k > 0:
            idx_cols.append(jc * float(_CW) + jl)
        if k < _K - 1:
            extm = jnp.where(iota_cw == jl, jnp.inf, ext)
            nm = jnp.min(extm, axis=1, keepdims=True)
            cm = jnp.where(iota_nc == jc, nm, cm)
            hist.append((jc, jl))

    idxs = jnp.concatenate(idx_cols, axis=1).astype(jnp.int32)

    gfull = lax.dot_general(fm_ref[0], wp_ref[...], (((1,), (0,)), ((), ())),
                            preferred_element_type=jnp.float32) + bp_ref[...]
    idx_ref[0] = idxs
    s4_ref[0] = s4
    v4p = jnp.concatenate([vtr, jnp.zeros((_R, 1), jnp.float32)], axis=1)
    v4_ref[0] = v4p
    for q in range(_NQ):
        g4_ref[q, 0] = gfull[:, q * _HQ:(q + 1) * _HQ]


def _tc_stage(vt, vertices, fm, wp, bp, a3p, c3p, nb):
    grid = (nb, _T)
    return pl.pallas_call(
        _tc_body,
        grid=grid,
        in_specs=[
            pl.BlockSpec((1, 3, _V), lambda b, t: (b, 0, 0)),
            pl.BlockSpec((1, _R, 3), lambda b, t: (b, t, 0)),
            pl.BlockSpec((1, _R, _IN), lambda b, t: (b, t, 0)),
            pl.BlockSpec((_IN, _OUT), lambda b, t: (0, 0)),
            pl.BlockSpec((1, _OUT), lambda b, t: (0, 0)),
            pl.BlockSpec((8, 128), lambda b, t: (0, 0)),
            pl.BlockSpec((8, 128), lambda b, t: (0, 0)),
        ],
        out_specs=[
            pl.BlockSpec((1, _R, _NS), lambda b, t: (b, t, 0)),
            pl.BlockSpec((1, _R, 4), lambda b, t: (b, t, 0)),
            pl.BlockSpec((1, _R, 4), lambda b, t: (b, t, 0)),
            pl.BlockSpec((_NQ, 1, _R, _HQ), lambda b, t: (0, b, t, 0)),
        ],
        out_shape=[
            jax.ShapeDtypeStruct((nb, _V, _NS), jnp.int32),
            jax.ShapeDtypeStruct((nb, _V, 4), jnp.float32),
            jax.ShapeDtypeStruct((nb, _V, 4), jnp.float32),
            jax.ShapeDtypeStruct((_NQ, nb, _V, _HQ), jnp.float32),
        ],
    )(vt, vertices, fm, wp, bp, a3p, c3p)


def _sc_body(g4, idxf, s4f, v4, out, gloc, idxloc, sloc, cloc, stage, sems):
    cid = lax.axis_index("c")
    sid = lax.axis_index("s")
    wid = sid * 2 + cid
    b = wid % _NB
    h = (wid // _NB) % _NQ
    vq = wid // (_NB * _NQ)
    v0 = vq * (_V // 4)
    row_base = b * _OUT + h * _HQ

    pltpu.sync_copy(g4.at[h * _NB + b], gloc)
    pltpu.sync_copy(v4.at[b], cloc)

    def chunk_body(ch, carry):
        slot = lax.rem(ch, 2)
        r0 = b * _V + v0 + ch * _VB
        pltpu.sync_copy(idxf.at[pl.ds(r0 * _NS, _VB * _NS)], idxloc)
        pltpu.sync_copy(s4f.at[pl.ds(r0 * 4, _VB * 4)],
                        sloc.at[pl.ds(0, _VB * 4)])

        @pl.when(ch >= 2)
        def _wait_slot():
            pltpu.make_async_copy(
                stage.at[slot],
                out.at[pl.ds(row_base, _HQ),
                       pl.ds((v0 + ch * _VB) * _NS, _VB * _NS)],
                sems.at[slot]).wait()

        @plsc.parallel_loop(0, _VB)
        def vb_body(vb):
            iv = idxloc[pl.ds(vb * _NS, _NS)]
            iv4 = iv * 4
            xn = plsc.load_gather(cloc, [iv4])
            yn = plsc.load_gather(cloc, [iv4 + 1])
            zn = plsc.load_gather(cloc, [iv4 + 2])
            sv = sloc[pl.ds(vb * 4, _NS)]
            lg = xn * sv[0] + yn * sv[1] + zn * sv[2]
            mx = jnp.max(lg)
            ex = jnp.exp((lg - mx) * (1.0 / math.sqrt(_BC)))
            tv = ex / jnp.sum(ex)
            ivh = iv * _HQ

            @plsc.parallel_loop(0, _HQ, unroll=8)
            def c_body(c):
                col = plsc.load_gather(gloc, [ivh + c])
                stage[slot, c, pl.ds(vb * _NS, _NS)] = col * tv

        pltpu.make_async_copy(
            stage.at[slot],
            out.at[pl.ds(row_base, _HQ),
                   pl.ds((v0 + ch * _VB) * _NS, _VB * _NS)],
            sems.at[slot]).start()
        return carry

    lax.fori_loop(0, _NCHUNK, chunk_body, 0)

    for slot in range(2):
        pltpu.make_async_copy(
            stage.at[slot],
            out.at[pl.ds(row_base, _HQ), pl.ds(0, _VB * _NS)],
            sems.at[slot]).wait()


def _sc_stage(g4, idxf, s4f, v4):
    mesh = plsc.VectorSubcoreMesh(core_axis_name="c", subcore_axis_name="s")
    k = functools.partial(
        pl.kernel,
        out_type=jax.ShapeDtypeStruct((_NB * _OUT, _V * _NS), jnp.float32),
        mesh=mesh,
        compiler_params=pltpu.CompilerParams(needs_layout_passes=False),
        scratch_types=[
            pltpu.VMEM((_V * _HQ,), jnp.float32),
            pltpu.VMEM((_VB * _NS,), jnp.int32),
            pltpu.VMEM((_VB * 4 + _NS,), jnp.float32),
            pltpu.VMEM((_V * 4,), jnp.float32),
            pltpu.VMEM((2, _HQ, _VB * _NS), jnp.float32),
            pltpu.SemaphoreType.DMA((2,)),
        ],
    )(_sc_body)
    return k(g4, idxf, s4f, v4)


def kernel(xyz, vertices, feature_map, weights, bias, Wkv, bkv, Wq, bq):
    wp = 0.5 * (weights[:, _OUT:2 * _OUT] + weights[:, 2 * _OUT:])
    bp = (0.5 * (bias[_OUT:2 * _OUT] + bias[2 * _OUT:])).reshape(1, _OUT)
    wk = Wkv[:_BC]
    a3 = jnp.transpose(wk.T @ Wq)
    c3 = (bq @ wk).reshape(1, 3)
    a3p = jnp.zeros((8, 128), jnp.float32).at[0:3, 0:3].set(a3)
    c3p = jnp.zeros((8, 128), jnp.float32).at[0:1, 0:3].set(c3)
    vt = jnp.transpose(vertices, (0, 2, 1))

    outs = []
    for sh in range(_BS // _NB):
        s0 = sh * _NB
        idx, s4, v4o, g4 = _tc_stage(
            vt[s0:s0 + _NB], vertices[s0:s0 + _NB],
            feature_map[s0:s0 + _NB], wp, bp, a3p, c3p, _NB)
        g4f = g4.reshape(_NQ * _NB, _V * _HQ)
        idxf = idx.reshape(_NB * _V * _NS)
        s4f = s4.reshape(_NB * _V * 4)
        v4 = v4o.reshape(_NB, _V * 4)
        outs.append(_sc_stage(g4f, idxf, s4f, v4))
    out = jnp.stack(outs)
    return out.reshape(_BS, _OUT, _V, _NS)


# speedup vs baseline: 18.4944x; 1.0466x over previous
import functools
import math

import jax
import jax.numpy as jnp
from jax import lax
from jax.experimental import pallas as pl
from jax.experimental.pallas import tpu as pltpu
from jax.experimental.pallas import tpu_sc as plsc

_BS, _V, _NS = 8, 2048, 16
_IN, _OUT, _SUP = 128, 64, 2
_BC = max(32, _IN // 2)
_R = 512
_T = _V // _R
_K = _NS + 1
_HQ = 16
_NQ = _OUT // _HQ
_NB = 1
_NVQ = 32 // (_NB * 4)
_VB = 128
_NCHUNK = (_V // _NVQ) // _VB


def _tc_body(vt_ref, vr_ref, fm_ref, wp_ref, bp_ref, a3_ref, c3_ref,
             idx_ref, s4_ref, v4_ref, g4_ref):
    vtf = vt_ref[0]
    vtr = vr_ref[0]

    inner = lax.dot_general(vtr, vtf, (((1,), (0,)), ((), ())),
                            preferred_element_type=jnp.float32)
    qc = jnp.sum(vtf * vtf, axis=0, keepdims=True)
    qr = jnp.sum(vtr * vtr, axis=1, keepdims=True)
    dist = (-2.0 * inner + qc) + qr

    a34 = a3_ref[0:3, 0:4]
    c34 = c3_ref[0:1, 0:4]
    s4 = lax.dot_general(vtr, a34, (((1,), (0,)), ((), ())),
                         preferred_element_type=jnp.float32) + c34

    _NC, _CW = 16, 128
    chunks = [dist[:, c * _CW:(c + 1) * _CW] for c in range(_NC)]
    cm = jnp.concatenate(
        [jnp.min(ch, axis=1, keepdims=True) for ch in chunks], axis=1)
    iota_nc = lax.broadcasted_iota(jnp.int32, (_R, _NC), 1).astype(jnp.float32)
    iota_cw = lax.broadcasted_iota(jnp.int32, (_R, _CW), 1).astype(jnp.float32)
    idx_cols = []
    hist = []
    for k in range(_K):
        m = jnp.min(cm, axis=1, keepdims=True)
        jc = jnp.min(jnp.where(cm == m, iota_nc, float(_NC)), axis=1,
                     keepdims=True)
        jci = jc.astype(jnp.int32)
        lvl = chunks
        for lbit in range(4):
            bit = ((jci >> lbit) & 1) == 1
            lvl = [jnp.where(bit, lvl[2 * i + 1], lvl[2 * i])
                   for i in range(len(lvl) // 2)]
        ext = lvl[0]
        for (pjc, pjl) in hist:
            ext = jnp.where((pjc == jc) & (iota_cw == pjl), jnp.inf, ext)
        jl = jnp.min(jnp.where(ext == m, iota_cw, float(_CW)), axis=1,
                     keepdims=True)
        if k > 0:
            idx_cols.append(jc * float(_CW) + jl)
        if k < _K - 1:
            extm = jnp.where(iota_cw == jl, jnp.inf, ext)
            nm = jnp.min(extm, axis=1, keepdims=True)
            cm = jnp.where(iota_nc == jc, nm, cm)
            hist.append((jc, jl))

    idxs = jnp.concatenate(idx_cols, axis=1).astype(jnp.int32)

    gfull = lax.dot_general(fm_ref[0], wp_ref[...], (((1,), (0,)), ((), ())),
                            preferred_element_type=jnp.float32) + bp_ref[...]
    idx_ref[0] = idxs
    s4_ref[0] = s4
    v4p = jnp.concatenate([vtr, jnp.zeros((_R, 1), jnp.float32)], axis=1)
    v4_ref[0] = v4p
    for q in range(_NQ):
        g4_ref[q, 0] = gfull[:, q * _HQ:(q + 1) * _HQ]


def _tc_stage(vt, vertices, fm, wp, bp, a3p, c3p, nb):
    grid = (nb, _T)
    return pl.pallas_call(
        _tc_body,
        grid=grid,
        in_specs=[
            pl.BlockSpec((1, 3, _V), lambda b, t: (b, 0, 0)),
            pl.BlockSpec((1, _R, 3), lambda b, t: (b, t, 0)),
            pl.BlockSpec((1, _R, _IN), lambda b, t: (b, t, 0)),
            pl.BlockSpec((_IN, _OUT), lambda b, t: (0, 0)),
            pl.BlockSpec((1, _OUT), lambda b, t: (0, 0)),
            pl.BlockSpec((8, 128), lambda b, t: (0, 0)),
            pl.BlockSpec((8, 128), lambda b, t: (0, 0)),
        ],
        out_specs=[
            pl.BlockSpec((1, _R, _NS), lambda b, t: (b, t, 0)),
            pl.BlockSpec((1, _R, 4), lambda b, t: (b, t, 0)),
            pl.BlockSpec((1, _R, 4), lambda b, t: (b, t, 0)),
            pl.BlockSpec((_NQ, 1, _R, _HQ), lambda b, t: (0, b, t, 0)),
        ],
        out_shape=[
            jax.ShapeDtypeStruct((nb, _V, _NS), jnp.int32),
            jax.ShapeDtypeStruct((nb, _V, 4), jnp.float32),
            jax.ShapeDtypeStruct((nb, _V, 4), jnp.float32),
            jax.ShapeDtypeStruct((_NQ, nb, _V, _HQ), jnp.float32),
        ],
    )(vt, vertices, fm, wp, bp, a3p, c3p)


def _sc_body(g4, idxf, s4f, v4, out, gloc, idxloc, sloc, cloc, stage, sems):
    cid = lax.axis_index("c")
    sid = lax.axis_index("s")
    wid = sid * 2 + cid
    b = wid % _NB
    h = (wid // _NB) % _NQ
    vq = wid // (_NB * _NQ)
    v0 = vq * (_V // _NVQ)
    row_base = b * _OUT + h * _HQ

    pltpu.sync_copy(g4.at[h * _NB + b], gloc)
    pltpu.sync_copy(v4.at[b], cloc)

    def chunk_body(ch, carry):
        slot = lax.rem(ch, 2)
        r0 = b * _V + v0 + ch * _VB
        pltpu.sync_copy(idxf.at[pl.ds(r0 * _NS, _VB * _NS)], idxloc)
        pltpu.sync_copy(s4f.at[pl.ds(r0 * 4, _VB * 4)],
                        sloc.at[pl.ds(0, _VB * 4)])

        @pl.when(ch >= 2)
        def _wait_slot():
            pltpu.make_async_copy(
                stage.at[slot],
                out.at[pl.ds(row_base, _HQ),
                       pl.ds((v0 + ch * _VB) * _NS, _VB * _NS)],
                sems.at[slot]).wait()

        @plsc.parallel_loop(0, _VB)
        def vb_body(vb):
            iv = idxloc[pl.ds(vb * _NS, _NS)]
            iv4 = iv * 4
            xn = plsc.load_gather(cloc, [iv4])
            yn = plsc.load_gather(cloc, [iv4 + 1])
            zn = plsc.load_gather(cloc, [iv4 + 2])
            sv = sloc[pl.ds(vb * 4, _NS)]
            lg = xn * sv[0] + yn * sv[1] + zn * sv[2]
            mx = jnp.max(lg)
            ex = jnp.exp((lg - mx) * (1.0 / math.sqrt(_BC)))
            tv = ex / jnp.sum(ex)
            ivh = iv * _HQ

            @plsc.parallel_loop(0, _HQ, unroll=8)
            def c_body(c):
                col = plsc.load_gather(gloc, [ivh + c])
                stage[slot, c, pl.ds(vb * _NS, _NS)] = col * tv

        pltpu.make_async_copy(
            stage.at[slot],
            out.at[pl.ds(row_base, _HQ),
                   pl.ds((v0 + ch * _VB) * _NS, _VB * _NS)],
            sems.at[slot]).start()
        return carry

    lax.fori_loop(0, _NCHUNK, chunk_body, 0)

    for slot in range(2):
        pltpu.make_async_copy(
            stage.at[slot],
            out.at[pl.ds(row_base, _HQ), pl.ds(0, _VB * _NS)],
            sems.at[slot]).wait()


def _sc_stage(g4, idxf, s4f, v4):
    mesh = plsc.VectorSubcoreMesh(core_axis_name="c", subcore_axis_name="s")
    k = functools.partial(
        pl.kernel,
        out_type=jax.ShapeDtypeStruct((_NB * _OUT, _V * _NS), jnp.float32),
        mesh=mesh,
        compiler_params=pltpu.CompilerParams(needs_layout_passes=False),
        scratch_types=[
            pltpu.VMEM((_V * _HQ,), jnp.float32),
            pltpu.VMEM((_VB * _NS,), jnp.int32),
            pltpu.VMEM((_VB * 4 + _NS,), jnp.float32),
            pltpu.VMEM((_V * 4,), jnp.float32),
            pltpu.VMEM((2, _HQ, _VB * _NS), jnp.float32),
            pltpu.SemaphoreType.DMA((2,)),
        ],
    )(_sc_body)
    return k(g4, idxf, s4f, v4)


def kernel(xyz, vertices, feature_map, weights, bias, Wkv, bkv, Wq, bq):
    wp = 0.5 * (weights[:, _OUT:2 * _OUT] + weights[:, 2 * _OUT:])
    bp = (0.5 * (bias[_OUT:2 * _OUT] + bias[2 * _OUT:])).reshape(1, _OUT)
    wk = Wkv[:_BC]
    a3 = jnp.transpose(wk.T @ Wq)
    c3 = (bq @ wk).reshape(1, 3)
    a3p = jnp.zeros((8, 128), jnp.float32).at[0:3, 0:3].set(a3)
    c3p = jnp.zeros((8, 128), jnp.float32).at[0:1, 0:3].set(c3)
    vt = jnp.transpose(vertices, (0, 2, 1))

    outs = []
    for sh in range(_BS // _NB):
        s0 = sh * _NB
        idx, s4, v4o, g4 = _tc_stage(
            vt[s0:s0 + _NB], vertices[s0:s0 + _NB],
            feature_map[s0:s0 + _NB], wp, bp, a3p, c3p, _NB)
        g4f = g4.reshape(_NQ * _NB, _V * _HQ)
        idxf = idx.reshape(_NB * _V * _NS)
        s4f = s4.reshape(_NB * _V * 4)
        v4 = v4o.reshape(_NB, _V * 4)
        outs.append(_sc_stage(g4f, idxf, s4f, v4))
    out = jnp.stack(outs)
    return out.reshape(_BS, _OUT, _V, _NS)
